# Initial kernel scaffold; baseline (speedup 1.0000x reference)
import jax
import jax.numpy as jnp
from jax.experimental import pallas as pl


def kernel(z, dictionary):
    raise NotImplementedError("write your pallas kernel here")


# speedup vs baseline: 106.6159x; 106.6159x over previous
import jax
import jax.numpy as jnp
from jax.experimental import pallas as pl

_EPS = 1e-10
_K_SPARSE = 5
_COMMIT = 0.25


def _body(x_ref, d_ref, dt_ref, xhat_ref, loss_ref):
    TB, C = x_ref.shape
    K = d_ref.shape[1]
    xb = x_ref[...]
    corr = jnp.dot(xb, d_ref[...], preferred_element_type=jnp.float32)
    a = jnp.abs(corr)
    iota = jax.lax.broadcasted_iota(jnp.int32, (TB, K), 1)
    atoms = []
    rhs = []
    for _ in range(_K_SPARSE):
        m = jnp.max(a, axis=1, keepdims=True)
        hit = a >= m
        idx = jnp.min(jnp.where(hit, iota, K), axis=1, keepdims=True)
        sel = iota == idx
        rhs.append(jnp.sum(jnp.where(sel, corr, 0.0), axis=1, keepdims=True))
        atoms.append(jnp.dot(sel.astype(jnp.float32), dt_ref[...],
                             preferred_element_type=jnp.float32))
        a = jnp.where(sel, -1.0, a)

    k = _K_SPARSE
    G = [[None] * k for _ in range(k)]
    for i in range(k):
        for j in range(i, k):
            g = jnp.sum(atoms[i] * atoms[j], axis=1, keepdims=True)
            if i == j:
                g = g + _EPS
            G[i][j] = g
            G[j][i] = g
    b = list(rhs)
    for p in range(k):
        inv = 1.0 / G[p][p]
        for r in range(p + 1, k):
            f = G[r][p] * inv
            for c2 in range(p + 1, k):
                G[r][c2] = G[r][c2] - f * G[p][c2]
            b[r] = b[r] - f * b[p]
    v = [None] * k
    for p in range(k - 1, -1, -1):
        acc = b[p]
        for c2 in range(p + 1, k):
            acc = acc - G[p][c2] * v[c2]
        v[p] = acc / G[p][p]
    v = [jnp.where(jnp.isfinite(val), val, 0.0) for val in v]

    xhat = v[0] * atoms[0]
    for i in range(1, k):
        xhat = xhat + v[i] * atoms[i]
    xhat_ref[...] = xhat
    d2 = xhat - xb
    s = jnp.sum(d2 * d2).reshape(1, 1)

    @pl.when(pl.program_id(0) == 0)
    def _():
        loss_ref[...] = jnp.zeros((1, 1), jnp.float32)

    loss_ref[...] += s


def kernel(z, dictionary):
    B, C, H, W = z.shape
    K = dictionary.shape[1]
    N = B * H * W
    X = jnp.transpose(z, (0, 2, 3, 1)).reshape(N, C)
    D = dictionary
    DT = D.T
    TB = 512
    nblk = N // TB

    xhat, loss_sum = pl.pallas_call(
        _body,
        grid=(nblk,),
        in_specs=[
            pl.BlockSpec((TB, C), lambda i: (i, 0)),
            pl.BlockSpec((C, K), lambda i: (0, 0)),
            pl.BlockSpec((K, C), lambda i: (0, 0)),
        ],
        out_specs=[
            pl.BlockSpec((TB, C), lambda i: (i, 0)),
            pl.BlockSpec((1, 1), lambda i: (0, 0)),
        ],
        out_shape=[
            jax.ShapeDtypeStruct((N, C), jnp.float32),
            jax.ShapeDtypeStruct((1, 1), jnp.float32),
        ],
    )(X, D, DT)

    quant = jnp.transpose(xhat.reshape(B, H, W, C), (0, 3, 1, 2))
    loss = loss_sum[0, 0] * (1.0 + _COMMIT) / (N * C)
    return quant, loss


# speedup vs baseline: 182.0202x; 1.7073x over previous
import jax
import jax.numpy as jnp
from jax.experimental import pallas as pl

_EPS = 1e-10
_K_SPARSE = 5
_COMMIT = 0.25


def _body(xt_ref, d_ref, xhat_ref, loss_ref):
    C, TB = xt_ref.shape
    K = d_ref.shape[1]
    xbt = xt_ref[...]
    dmat = d_ref[...]
    corr = jax.lax.dot_general(
        dmat, xbt, (((0,), (0,)), ((), ())),
        preferred_element_type=jnp.float32)
    a = jnp.abs(corr)
    atoms = []
    rhs = []
    for _ in range(_K_SPARSE):
        m = jnp.max(a, axis=0, keepdims=True)
        hit = a >= m
        rhs.append(jnp.sum(jnp.where(hit, corr, 0.0), axis=0, keepdims=True))
        atoms.append(jax.lax.dot_general(
            dmat, hit.astype(jnp.float32), (((1,), (0,)), ((), ())),
            preferred_element_type=jnp.float32))
        a = jnp.where(hit, -1.0, a)

    k = _K_SPARSE
    G = [[None] * k for _ in range(k)]
    for i in range(k):
        for j in range(i, k):
            g = jnp.sum(atoms[i] * atoms[j], axis=0, keepdims=True)
            if i == j:
                g = g + _EPS
            G[i][j] = g
            G[j][i] = g
    b = list(rhs)
    for p in range(k):
        inv = 1.0 / G[p][p]
        for r in range(p + 1, k):
            f = G[r][p] * inv
            for c2 in range(p + 1, k):
                G[r][c2] = G[r][c2] - f * G[p][c2]
            b[r] = b[r] - f * b[p]
    v = [None] * k
    for p in range(k - 1, -1, -1):
        acc = b[p]
        for c2 in range(p + 1, k):
            acc = acc - G[p][c2] * v[c2]
        v[p] = acc / G[p][p]
    v = [jnp.where(jnp.isfinite(val), val, 0.0) for val in v]

    xhat = v[0] * atoms[0]
    for i in range(1, k):
        xhat = xhat + v[i] * atoms[i]
    xhat_ref[...] = xhat
    d2 = xhat - xbt
    s = jnp.sum(d2 * d2).reshape(1, 1)

    @pl.when(pl.program_id(0) == 0)
    def _():
        loss_ref[...] = jnp.zeros((1, 1), jnp.float32)

    loss_ref[...] += s


def kernel(z, dictionary):
    B, C, H, W = z.shape
    K = dictionary.shape[1]
    N = B * H * W
    XT = jnp.transpose(z, (1, 0, 2, 3)).reshape(C, N)
    TB = 512
    nblk = N // TB

    xhat_t, loss_sum = pl.pallas_call(
        _body,
        grid=(nblk,),
        in_specs=[
            pl.BlockSpec((C, TB), lambda i: (0, i)),
            pl.BlockSpec((C, K), lambda i: (0, 0)),
        ],
        out_specs=[
            pl.BlockSpec((C, TB), lambda i: (0, i)),
            pl.BlockSpec((1, 1), lambda i: (0, 0)),
        ],
        out_shape=[
            jax.ShapeDtypeStruct((C, N), jnp.float32),
            jax.ShapeDtypeStruct((1, 1), jnp.float32),
        ],
    )(XT, dictionary)

    quant = jnp.transpose(xhat_t.reshape(C, B, H, W), (1, 0, 2, 3))
    loss = loss_sum[0, 0] * (1.0 + _COMMIT) / (N * C)
    return quant, loss


# speedup vs baseline: 221.6727x; 1.2178x over previous
import jax
import jax.numpy as jnp
from jax.experimental import pallas as pl

_EPS = 1e-10
_K_SPARSE = 5
_COMMIT = 0.25


def _body(xt_ref, d_ref, xhat_ref, loss_ref):
    C, TB = xt_ref.shape
    K = d_ref.shape[1]
    xbt = xt_ref[...]
    dmat = d_ref[...]
    corr = jax.lax.dot_general(
        dmat, xbt, (((0,), (0,)), ((), ())),
        preferred_element_type=jnp.float32)
    a = jnp.abs(corr)
    atoms = []
    for _ in range(_K_SPARSE):
        m = jnp.max(a, axis=0, keepdims=True)
        hit = a >= m
        atoms.append(jax.lax.dot_general(
            dmat, hit.astype(jnp.float32), (((1,), (0,)), ((), ())),
            preferred_element_type=jnp.float32))
        a = jnp.where(hit, -1.0, a)
    rhs = [jnp.sum(at * xbt, axis=0, keepdims=True) for at in atoms]

    k = _K_SPARSE
    G = [[None] * k for _ in range(k)]
    for i in range(k):
        for j in range(i, k):
            g = jnp.sum(atoms[i] * atoms[j], axis=0, keepdims=True)
            if i == j:
                g = g + _EPS
            G[i][j] = g
            G[j][i] = g
    b = list(rhs)
    for p in range(k):
        inv = 1.0 / G[p][p]
        for r in range(p + 1, k):
            f = G[r][p] * inv
            for c2 in range(p + 1, k):
                G[r][c2] = G[r][c2] - f * G[p][c2]
            b[r] = b[r] - f * b[p]
    v = [None] * k
    for p in range(k - 1, -1, -1):
        acc = b[p]
        for c2 in range(p + 1, k):
            acc = acc - G[p][c2] * v[c2]
        v[p] = acc / G[p][p]
    v = [jnp.where(jnp.isfinite(val), val, 0.0) for val in v]

    xhat = v[0] * atoms[0]
    for i in range(1, k):
        xhat = xhat + v[i] * atoms[i]
    xhat_ref[...] = xhat
    d2 = xhat - xbt
    s = jnp.sum(d2 * d2).reshape(1, 1)

    @pl.when(pl.program_id(0) == 0)
    def _():
        loss_ref[...] = jnp.zeros((1, 1), jnp.float32)

    loss_ref[...] += s


def kernel(z, dictionary):
    B, C, H, W = z.shape
    K = dictionary.shape[1]
    N = B * H * W
    XT = jnp.transpose(z, (1, 0, 2, 3)).reshape(C, N)
    TB = 1024
    nblk = N // TB

    xhat_t, loss_sum = pl.pallas_call(
        _body,
        grid=(nblk,),
        in_specs=[
            pl.BlockSpec((C, TB), lambda i: (0, i)),
            pl.BlockSpec((C, K), lambda i: (0, 0)),
        ],
        out_specs=[
            pl.BlockSpec((C, TB), lambda i: (0, i)),
            pl.BlockSpec((1, 1), lambda i: (0, 0)),
        ],
        out_shape=[
            jax.ShapeDtypeStruct((C, N), jnp.float32),
            jax.ShapeDtypeStruct((1, 1), jnp.float32),
        ],
    )(XT, dictionary)

    quant = jnp.transpose(xhat_t.reshape(C, B, H, W), (1, 0, 2, 3))
    loss = loss_sum[0, 0] * (1.0 + _COMMIT) / (N * C)
    return quant, loss
